# baseline (speedup 1.0000x reference)
import jax
import jax.numpy as jnp
from jax import lax
from jax.experimental import pallas as pl
from jax.experimental.pallas import tpu as pltpu
from jax.experimental.pallas import tpu_sc as plsc

B, T, V, D, H, F = 1024, 50, 1000, 512, 512, 2048
G = 4 * H
GP = G // 2
OUT_PAD = 128

_NC, _NS = 2, 16
_NW = _NC * _NS
_ROWS_PER_W = (T * B) // _NW
_CHUNK = 64
_NCHUNK = _ROWS_PER_W // _CHUNK


def _proj_body(emb_ref, w_ref, b_ref, out_ref):
    out_ref[...] = (
        jnp.dot(emb_ref[...], w_ref[...], preferred_element_type=jnp.float32)
        + b_ref[...]
    ).astype(jnp.bfloat16)


def _gather_body(table_hbm, idx_hbm, out_hbm, idx_v, rows_v, sem):
    wid = lax.axis_index("s") * _NC + lax.axis_index("c")
    base = wid * _ROWS_PER_W
    pltpu.sync_copy(idx_hbm.at[pl.ds(base, _ROWS_PER_W)], idx_v)

    def chunk(ch, carry):
        r0 = ch * _CHUNK
        pltpu.async_copy(
            table_hbm.at[idx_v.at[pl.ds(r0, _CHUNK)]], rows_v, sem
        ).wait()
        pltpu.sync_copy(rows_v, out_hbm.at[pl.ds(base + r0, _CHUNK)])
        return carry

    lax.fori_loop(0, _NCHUNK, chunk, 0)


def _lstm_body(x_ref, whh_ref, w1_ref, b1_ref, w2_ref, b2_ref,
               out_ref, h_ref, c_ref):
    t = pl.program_id(0)

    @pl.when(t == 0)
    def _():
        h_ref[...] = jnp.zeros_like(h_ref)
        c_ref[...] = jnp.zeros_like(c_ref)

    gates = x_ref[0].astype(jnp.float32) + jnp.dot(
        h_ref[...], whh_ref[...], preferred_element_type=jnp.float32
    )
    i = jax.nn.sigmoid(gates[:, 0:H])
    f = jax.nn.sigmoid(gates[:, H:2 * H])
    g = jnp.tanh(gates[:, 2 * H:3 * H])
    o = jax.nn.sigmoid(gates[:, 3 * H:4 * H])
    c_new = f * c_ref[...] + i * g
    h_new = o * jnp.tanh(c_new)
    c_ref[...] = c_new
    h_ref[...] = h_new.astype(jnp.bfloat16)

    @pl.when(t == T - 1)
    def _():
        a = jnp.maximum(
            jnp.dot(h_new.astype(jnp.bfloat16), w1_ref[...],
                    preferred_element_type=jnp.float32)
            + b1_ref[...],
            0.0,
        ).astype(jnp.bfloat16)
        out_ref[...] = (
            jnp.dot(a, w2_ref[...], preferred_element_type=jnp.float32)
            + b2_ref[...]
        )


def kernel(src_seq, src_pos, emb, W_ih, W_hh, b_ih, b_hh, W1, b1, W2, b2):
    bias = (b_ih + b_hh).reshape(1, G)
    P = pl.pallas_call(
        _proj_body,
        out_shape=jax.ShapeDtypeStruct((V, G), jnp.bfloat16),
    )(emb, W_ih.T, bias)
    P_packed = lax.bitcast_convert_type(P.reshape(V, GP, 2), jnp.int32)

    flat_idx = src_seq.T.reshape(T * B).astype(jnp.int32)
    gather = pl.kernel(
        _gather_body,
        out_type=jax.ShapeDtypeStruct((T * B, GP), jnp.int32),
        mesh=plsc.VectorSubcoreMesh(core_axis_name="c", subcore_axis_name="s"),
        scratch_types=[
            pltpu.VMEM((_ROWS_PER_W,), jnp.int32),
            pltpu.VMEM((_CHUNK, GP), jnp.int32),
            pltpu.SemaphoreType.DMA,
        ],
    )
    Xp = gather(P_packed, flat_idx)
    X = lax.bitcast_convert_type(Xp, jnp.bfloat16).reshape(T, B, G)

    W2p = jnp.pad(W2.T, ((0, 0), (0, OUT_PAD - 2))).astype(jnp.bfloat16)
    b2p = jnp.pad(b2, (0, OUT_PAD - 2)).reshape(1, OUT_PAD)

    out_p = pl.pallas_call(
        _lstm_body,
        grid=(T,),
        in_specs=[
            pl.BlockSpec((1, B, G), lambda t: (t, 0, 0)),
            pl.BlockSpec((H, G), lambda t: (0, 0)),
            pl.BlockSpec((H, F), lambda t: (0, 0)),
            pl.BlockSpec((1, F), lambda t: (0, 0)),
            pl.BlockSpec((F, OUT_PAD), lambda t: (0, 0)),
            pl.BlockSpec((1, OUT_PAD), lambda t: (0, 0)),
        ],
        out_specs=pl.BlockSpec((B, OUT_PAD), lambda t: (0, 0)),
        out_shape=jax.ShapeDtypeStruct((B, OUT_PAD), jnp.float32),
        scratch_shapes=[
            pltpu.VMEM((B, H), jnp.bfloat16),
            pltpu.VMEM((B, H), jnp.float32),
        ],
    )(X, W_hh.T.astype(jnp.bfloat16), W1.T.astype(jnp.bfloat16),
      b1.reshape(1, F), W2p, b2p)
    return out_p[:, :2]


# speedup vs baseline: 6.1534x; 6.1534x over previous
import jax
import jax.numpy as jnp
from jax import lax
from jax.experimental import pallas as pl
from jax.experimental.pallas import tpu as pltpu
from jax.experimental.pallas import tpu_sc as plsc

B, T, V, D, H, F = 1024, 50, 1000, 512, 512, 2048
G = 4 * H
GP = G // 2
OUT_PAD = 128

_NC, _NS = 2, 16
_NW = _NC * _NS
_ROWS_PER_W = (T * B) // _NW
_CHUNK = 64
_NCHUNK = _ROWS_PER_W // _CHUNK


def _proj_body(emb_ref, w_ref, b_ref, out_ref):
    p = (
        jnp.dot(emb_ref[...], w_ref[...], preferred_element_type=jnp.float32)
        + b_ref[...]
    )
    lo_bits = lax.bitcast_convert_type(p[:, :GP], jnp.int32) + 0x8000
    hi_bits = lax.bitcast_convert_type(p[:, GP:], jnp.int32) + 0x8000
    lo16 = lax.shift_right_logical(lo_bits, 16)
    hi16 = hi_bits & jnp.int32(-65536)
    out_ref[...] = hi16 | lo16


def _gather_body(table_hbm, idx_hbm, out_hbm, idx_v, rows_v, sem):
    wid = lax.axis_index("s") * _NC + lax.axis_index("c")
    base = wid * _ROWS_PER_W
    pltpu.sync_copy(idx_hbm.at[pl.ds(base, _ROWS_PER_W)], idx_v)

    def chunk(ch, carry):
        r0 = ch * _CHUNK
        pltpu.async_copy(
            table_hbm.at[idx_v.at[pl.ds(r0, _CHUNK)]], rows_v, sem
        ).wait()
        pltpu.sync_copy(rows_v, out_hbm.at[pl.ds(base + r0, _CHUNK)])
        return carry

    lax.fori_loop(0, _NCHUNK, chunk, 0)


def _lstm_body(x_ref, whh_ref, w1_ref, b1_ref, w2_ref, b2_ref,
               out_ref, h_ref, c_ref):
    t = pl.program_id(0)

    @pl.when(t == 0)
    def _():
        h_ref[...] = jnp.zeros_like(h_ref)
        c_ref[...] = jnp.zeros_like(c_ref)

    w = x_ref[0]
    x_lo = lax.bitcast_convert_type(lax.shift_left(w, 16), jnp.float32)
    x_hi = lax.bitcast_convert_type(w & jnp.int32(-65536), jnp.float32)
    hw = jnp.dot(h_ref[...], whh_ref[...], preferred_element_type=jnp.float32)
    i = jax.nn.sigmoid(x_lo[:, 0:H] + hw[:, 0:H])
    f = jax.nn.sigmoid(x_lo[:, H:2 * H] + hw[:, H:2 * H])
    g = jnp.tanh(x_hi[:, 0:H] + hw[:, 2 * H:3 * H])
    o = jax.nn.sigmoid(x_hi[:, H:2 * H] + hw[:, 3 * H:4 * H])
    c_new = f * c_ref[...] + i * g
    h_new = o * jnp.tanh(c_new)
    c_ref[...] = c_new
    h_ref[...] = h_new.astype(jnp.bfloat16)

    @pl.when(t == T - 1)
    def _():
        a = jnp.maximum(
            jnp.dot(h_new.astype(jnp.bfloat16), w1_ref[...],
                    preferred_element_type=jnp.float32)
            + b1_ref[...],
            0.0,
        ).astype(jnp.bfloat16)
        out_ref[...] = (
            jnp.dot(a, w2_ref[...], preferred_element_type=jnp.float32)
            + b2_ref[...]
        )


def kernel(src_seq, src_pos, emb, W_ih, W_hh, b_ih, b_hh, W1, b1, W2, b2):
    bias = (b_ih + b_hh).reshape(1, G)
    P_packed = pl.pallas_call(
        _proj_body,
        out_shape=jax.ShapeDtypeStruct((V, GP), jnp.int32),
    )(emb, W_ih.T, bias)

    flat_idx = src_seq.T.reshape(T * B).astype(jnp.int32)
    gather = pl.kernel(
        _gather_body,
        out_type=jax.ShapeDtypeStruct((T * B, GP), jnp.int32),
        mesh=plsc.VectorSubcoreMesh(core_axis_name="c", subcore_axis_name="s"),
        scratch_types=[
            pltpu.VMEM((_ROWS_PER_W,), jnp.int32),
            pltpu.VMEM((_CHUNK, GP), jnp.int32),
            pltpu.SemaphoreType.DMA,
        ],
    )
    X = gather(P_packed, flat_idx).reshape(T, B, GP)

    W2p = jnp.pad(W2.T, ((0, 0), (0, OUT_PAD - 2))).astype(jnp.bfloat16)
    b2p = jnp.pad(b2, (0, OUT_PAD - 2)).reshape(1, OUT_PAD)

    out_p = pl.pallas_call(
        _lstm_body,
        grid=(T,),
        in_specs=[
            pl.BlockSpec((1, B, GP), lambda t: (t, 0, 0)),
            pl.BlockSpec((H, G), lambda t: (0, 0)),
            pl.BlockSpec((H, F), lambda t: (0, 0)),
            pl.BlockSpec((1, F), lambda t: (0, 0)),
            pl.BlockSpec((F, OUT_PAD), lambda t: (0, 0)),
            pl.BlockSpec((1, OUT_PAD), lambda t: (0, 0)),
        ],
        out_specs=pl.BlockSpec((B, OUT_PAD), lambda t: (0, 0)),
        out_shape=jax.ShapeDtypeStruct((B, OUT_PAD), jnp.float32),
        scratch_shapes=[
            pltpu.VMEM((B, H), jnp.bfloat16),
            pltpu.VMEM((B, H), jnp.float32),
        ],
    )(X, W_hh.T.astype(jnp.bfloat16), W1.T.astype(jnp.bfloat16),
      b1.reshape(1, F), W2p, b2p)
    return out_p[:, :2]


# speedup vs baseline: 7.1987x; 1.1699x over previous
import jax
import jax.numpy as jnp
from jax import lax
from jax.experimental import pallas as pl
from jax.experimental.pallas import tpu as pltpu
from jax.experimental.pallas import tpu_sc as plsc

B, T, V, D, H, F = 1024, 50, 1000, 512, 512, 2048
G = 4 * H
GP = G // 2
OUT_PAD = 128

TCH = 10
NCH = T // TCH

_NC, _NS = 2, 16
_NW = _NC * _NS
_ROWS_PER_W = (TCH * B) // _NW
_CHUNK = 64
_NCHUNK = _ROWS_PER_W // _CHUNK


def _proj_body(emb_ref, w_ref, b_ref, out_ref):
    p = (
        jnp.dot(emb_ref[...], w_ref[...], preferred_element_type=jnp.float32)
        + b_ref[...]
    )
    lo_bits = lax.bitcast_convert_type(p[:, :GP], jnp.int32) + 0x8000
    hi_bits = lax.bitcast_convert_type(p[:, GP:], jnp.int32) + 0x8000
    lo16 = lax.shift_right_logical(lo_bits, 16)
    hi16 = hi_bits & jnp.int32(-65536)
    out_ref[...] = hi16 | lo16


def _gather_body(table_hbm, idx_hbm, out_hbm, idx_v, rows_v, sem):
    wid = lax.axis_index("s") * _NC + lax.axis_index("c")
    base = wid * _ROWS_PER_W
    pltpu.sync_copy(idx_hbm.at[pl.ds(base, _ROWS_PER_W)], idx_v)

    def chunk(ch, carry):
        r0 = ch * _CHUNK
        pltpu.async_copy(
            table_hbm.at[idx_v.at[pl.ds(r0, _CHUNK)]], rows_v, sem
        ).wait()
        pltpu.sync_copy(rows_v, out_hbm.at[pl.ds(base + r0, _CHUNK)])
        return carry

    lax.fori_loop(0, _NCHUNK, chunk, 0)


def _lstm_body(x_ref, whh_ref, h_in_ref, c_in_ref, h_ref, c_ref):
    t = pl.program_id(0)

    @pl.when(t == 0)
    def _():
        h_ref[...] = h_in_ref[...]
        c_ref[...] = c_in_ref[...]

    w = x_ref[0]
    x_lo = lax.bitcast_convert_type(lax.shift_left(w, 16), jnp.float32)
    x_hi = lax.bitcast_convert_type(w & jnp.int32(-65536), jnp.float32)
    hw = jnp.dot(h_ref[...], whh_ref[...], preferred_element_type=jnp.float32)
    i = jax.nn.sigmoid(x_lo[:, 0:H] + hw[:, 0:H])
    f = jax.nn.sigmoid(x_lo[:, H:2 * H] + hw[:, H:2 * H])
    g = jnp.tanh(x_hi[:, 0:H] + hw[:, 2 * H:3 * H])
    o = jax.nn.sigmoid(x_hi[:, H:2 * H] + hw[:, 3 * H:4 * H])
    c_new = f * c_ref[...] + i * g
    h_new = o * jnp.tanh(c_new)
    c_ref[...] = c_new
    h_ref[...] = h_new.astype(jnp.bfloat16)


def _mlp_body(h_ref, w1_ref, b1_ref, w2_ref, b2_ref, out_ref):
    a = jnp.maximum(
        jnp.dot(h_ref[...], w1_ref[...], preferred_element_type=jnp.float32)
        + b1_ref[...],
        0.0,
    ).astype(jnp.bfloat16)
    out_ref[...] = (
        jnp.dot(a, w2_ref[...], preferred_element_type=jnp.float32)
        + b2_ref[...]
    )


def kernel(src_seq, src_pos, emb, W_ih, W_hh, b_ih, b_hh, W1, b1, W2, b2):
    bias = (b_ih + b_hh).reshape(1, G)
    P_packed = pl.pallas_call(
        _proj_body,
        out_shape=jax.ShapeDtypeStruct((V, GP), jnp.int32),
    )(emb, W_ih.T, bias)

    flat_idx = src_seq.T.reshape(NCH, TCH * B).astype(jnp.int32)
    gather = pl.kernel(
        _gather_body,
        out_type=jax.ShapeDtypeStruct((TCH * B, GP), jnp.int32),
        mesh=plsc.VectorSubcoreMesh(core_axis_name="c", subcore_axis_name="s"),
        scratch_types=[
            pltpu.VMEM((_ROWS_PER_W,), jnp.int32),
            pltpu.VMEM((_CHUNK, GP), jnp.int32),
            pltpu.SemaphoreType.DMA,
        ],
    )

    whh_bf = W_hh.T.astype(jnp.bfloat16)
    lstm_chunk = pl.pallas_call(
        _lstm_body,
        grid=(TCH,),
        in_specs=[
            pl.BlockSpec((1, B, GP), lambda t: (t, 0, 0)),
            pl.BlockSpec((H, G), lambda t: (0, 0)),
            pl.BlockSpec((B, H), lambda t: (0, 0)),
            pl.BlockSpec((B, H), lambda t: (0, 0)),
        ],
        out_specs=[
            pl.BlockSpec((B, H), lambda t: (0, 0)),
            pl.BlockSpec((B, H), lambda t: (0, 0)),
        ],
        out_shape=[
            jax.ShapeDtypeStruct((B, H), jnp.bfloat16),
            jax.ShapeDtypeStruct((B, H), jnp.float32),
        ],
    )

    h = jnp.zeros((B, H), jnp.bfloat16)
    c = jnp.zeros((B, H), jnp.float32)
    for k in range(NCH):
        X_k = gather(P_packed, flat_idx[k]).reshape(TCH, B, GP)
        h, c = lstm_chunk(X_k, whh_bf, h, c)

    W2p = jnp.pad(W2.T, ((0, 0), (0, OUT_PAD - 2))).astype(jnp.bfloat16)
    b2p = jnp.pad(b2, (0, OUT_PAD - 2)).reshape(1, OUT_PAD)
    out_p = pl.pallas_call(
        _mlp_body,
        out_shape=jax.ShapeDtypeStruct((B, OUT_PAD), jnp.float32),
    )(h, W1.T.astype(jnp.bfloat16), b1.reshape(1, F), W2p, b2p)
    return out_p[:, :2]
